# Initial kernel scaffold; baseline (speedup 1.0000x reference)
import jax
import jax.numpy as jnp
from jax.experimental import pallas as pl


def kernel(nodes, edges, edge_attr, batch, enc_W, enc_b, edge_W, edge_b, par_W1, par_b1, par_W2, par_b2, chi_W1, chi_b1, chi_W2, chi_b2, fin_W1, fin_b1, fin_W2, fin_b2, conv_W, conv_b):
    raise NotImplementedError("write your pallas kernel here")


# speedup vs baseline: 1.7237x; 1.7237x over previous
import functools

import jax
import jax.numpy as jnp
from jax import lax
from jax.experimental import pallas as pl
from jax.experimental.pallas import tpu as pltpu
from jax.experimental.pallas import tpu_sc as plsc

N = 10000
E = 320000
D = 128
B = 64
NUM_IT = 2

NC = 2
NS = 16
LANES = 16

EPT = 20480
EP = NS * EPT
KB = 64
NBLK = EPT // KB
SROWS = 10112


def _enc_body(x_ref, w_ref, b_ref, o_ref):
    o_ref[...] = jnp.dot(x_ref[...], w_ref[...],
                         preferred_element_type=jnp.float32) + b_ref[...]


def _enc(nodes, encWT, enc_b2):
    bn = 2000
    return pl.pallas_call(
        _enc_body,
        grid=(N // bn,),
        in_specs=[
            pl.BlockSpec((bn, D), lambda i: (i, 0)),
            pl.BlockSpec((D, D), lambda i: (0, 0)),
            pl.BlockSpec((1, D), lambda i: (0, 0)),
        ],
        out_specs=pl.BlockSpec((bn, D), lambda i: (i, 0)),
        out_shape=jax.ShapeDtypeStruct((N, D), jnp.float32),
    )(nodes, encWT, enc_b2)


def _stage_a_body(x_ref, wi_ref, wj_ref, c_ref, g_ref, h_ref):
    xb = x_ref[...]
    g_ref[0] = jnp.dot(xb, wi_ref[0], preferred_element_type=jnp.float32) + c_ref[0]
    h_ref[0] = jnp.dot(xb, wj_ref[0], preferred_element_type=jnp.float32)


def _stage_a(x, WiT, WjT, c_stack):
    bn = 2000
    G, H = pl.pallas_call(
        _stage_a_body,
        grid=(2, N // bn),
        in_specs=[
            pl.BlockSpec((bn, D), lambda a, i: (i, 0)),
            pl.BlockSpec((1, D, D), lambda a, i: (a, 0, 0)),
            pl.BlockSpec((1, D, D), lambda a, i: (a, 0, 0)),
            pl.BlockSpec((1, 1, D), lambda a, i: (a, 0, 0)),
        ],
        out_specs=[
            pl.BlockSpec((1, bn, D), lambda a, i: (a, i, 0)),
            pl.BlockSpec((1, bn, D), lambda a, i: (a, i, 0)),
        ],
        out_shape=[
            jax.ShapeDtypeStruct((2, N, D), jnp.float32),
            jax.ShapeDtypeStruct((2, N, D), jnp.float32),
        ],
    )(x, WiT, WjT, c_stack)
    return G.reshape(2 * N, D), H.reshape(2 * N, D)


def _stage_b_body(x_ref, sp_ref, sc_ref, deg_ref,
                  pw2_ref, pb2_ref, cw2_ref, cb2_ref,
                  f1x_ref, f1i_ref, f1o_ref, fb1_ref, f2_ref, fb2_ref,
                  o_ref):
    ds_ = deg_ref[:, 0:1]
    dd_ = deg_ref[:, 1:2]
    dinv = jnp.where(ds_ > 0, 1.0 / ds_, 0.0)
    fi = dinv * (jnp.dot(sp_ref[...], pw2_ref[...],
                         preferred_element_type=jnp.float32) + dd_ * pb2_ref[...])
    fo = dinv * (jnp.dot(sc_ref[...], cw2_ref[...],
                         preferred_element_type=jnp.float32) + ds_ * cb2_ref[...])
    xb = x_ref[...]
    h1 = jnp.dot(xb, f1x_ref[...], preferred_element_type=jnp.float32)
    h1 = h1 + jnp.dot(fi, f1i_ref[...], preferred_element_type=jnp.float32)
    h1 = h1 + jnp.dot(fo, f1o_ref[...], preferred_element_type=jnp.float32)
    h1 = jnp.maximum(h1 + fb1_ref[...], 0.0)
    o_ref[...] = xb + jnp.dot(h1, f2_ref[...],
                              preferred_element_type=jnp.float32) + fb2_ref[...]


def _stage_b(x, S, degT, pW2T, pb2, cW2T, cb2, F1xT, F1iT, F1oT, fb1, F2T, fb2):
    bn = 1000
    nb = N // bn
    full = lambda r, c: pl.BlockSpec((r, c), lambda i: (0, 0))
    return pl.pallas_call(
        _stage_b_body,
        grid=(nb,),
        in_specs=[
            pl.BlockSpec((bn, D), lambda i: (i, 0)),
            pl.BlockSpec((bn, D), lambda i: (i, 0)),
            pl.BlockSpec((bn, D), lambda i: (nb + i, 0)),
            pl.BlockSpec((bn, 2), lambda i: (i, 0)),
            full(D, D), full(1, D), full(D, D), full(1, D),
            full(D, 2 * D), full(D, 2 * D), full(D, 2 * D), full(1, 2 * D),
            full(2 * D, D), full(1, D),
        ],
        out_specs=pl.BlockSpec((bn, D), lambda i: (i, 0)),
        out_shape=jax.ShapeDtypeStruct((N, D), jnp.float32),
    )(x, S, S, degT, pW2T, pb2, cW2T, cb2, F1xT, F1iT, F1oT, fb1, F2T, fb2)


def _stage_c_body(x_ref, w_ref, b_ref, batch_ref, o_ref, gmax_s, gsum_s, cnt_s):
    i = pl.program_id(0)
    nb = pl.num_programs(0)

    @pl.when(i == 0)
    def _init():
        gmax_s[...] = jnp.full_like(gmax_s[...], -1e30)
        gsum_s[...] = jnp.zeros_like(gsum_s[...])
        cnt_s[...] = jnp.zeros_like(cnt_s[...])

    y = jnp.dot(x_ref[...], w_ref[...],
                preferred_element_type=jnp.float32) + b_ref[...]
    bb = batch_ref[...]
    seg = lax.broadcasted_iota(jnp.int32, (1, B), 1)
    onehot = (bb == seg).astype(jnp.float32)
    gsum_s[...] += lax.dot_general(onehot, y, (((0,), (0,)), ((), ())),
                                   preferred_element_type=jnp.float32)
    ones = jnp.ones_like(bb, jnp.float32)
    cnt_s[...] += lax.dot_general(onehot, ones, (((0,), (0,)), ((), ())),
                                  preferred_element_type=jnp.float32)
    bias = (onehot - 1.0) * 1e30
    for s in range(B):
        m = jnp.max(y + bias[:, s:s + 1], axis=0, keepdims=True)
        gmax_s[s:s + 1, :] = jnp.maximum(gmax_s[s:s + 1, :], m)

    @pl.when(i == nb - 1)
    def _fin():
        o_ref[:, :2 * D] = gmax_s[...]
        o_ref[:, 2 * D:] = gsum_s[...] / jnp.maximum(cnt_s[...], 1.0)


def _stage_c(x, convWT, conv_b2, batch2):
    bn = 1000
    return pl.pallas_call(
        _stage_c_body,
        grid=(N // bn,),
        in_specs=[
            pl.BlockSpec((bn, D), lambda i: (i, 0)),
            pl.BlockSpec((D, 2 * D), lambda i: (0, 0)),
            pl.BlockSpec((1, 2 * D), lambda i: (0, 0)),
            pl.BlockSpec((bn, 1), lambda i: (i, 0)),
        ],
        out_specs=pl.BlockSpec((B, 4 * D), lambda i: (0, 0)),
        out_shape=jax.ShapeDtypeStruct((B, 4 * D), jnp.float32),
        scratch_shapes=[
            pltpu.VMEM((B, 2 * D), jnp.float32),
            pltpu.VMEM((B, 2 * D), jnp.float32),
            pltpu.VMEM((B, 1), jnp.float32),
        ],
        compiler_params=pltpu.CompilerParams(
            dimension_semantics=("arbitrary",)),
    )(x, convWT, conv_b2, batch2)


_SC_MESH = plsc.VectorSubcoreMesh(core_axis_name="c", subcore_axis_name="s",
                                  num_cores=NC, num_subcores=NS)


_RPT = SROWS // NS


def _zero_stripe(src_v, sh, t):
    _zfull, _zrem = _RPT // KB, _RPT % KB
    for r in range(_zfull):
        pltpu.sync_copy(src_v, sh.at[pl.ds(t * _RPT + r * KB, KB)])
    if _zrem:
        pltpu.sync_copy(src_v.at[pl.ds(0, _zrem)],
                        sh.at[pl.ds(t * _RPT + _zfull * KB, _zrem)])


def _copy_out_stripe(sh, out_ref, t, base):
    cw = 632
    rem = N - (NS - 1) * cw

    @pl.when(t < NS - 1)
    def _cp():
        pltpu.sync_copy(sh.at[pl.ds(t * cw, cw)],
                        out_ref.at[pl.ds(base + t * cw, cw)])

    @pl.when(t == NS - 1)
    def _cp_last():
        pltpu.sync_copy(sh.at[pl.ds((NS - 1) * cw, rem)],
                        out_ref.at[pl.ds(base + (NS - 1) * cw, rem)])


def _edge_kernel(gt_ref, ht_ref, gI_ref, gJ_ref, sI_ref, attr_ref, u_ref,
                 out_ref, u_v, gi_v, hj_v, gIv, gJv, sIv, atv,
                 s_sh, sem1, sem2):
    a = lax.axis_index("c")
    t = lax.axis_index("s")

    pltpu.sync_copy(u_ref.at[pl.ds(a * D, D)], u_v)
    z16 = jnp.zeros((LANES,), jnp.float32)

    def zb_body(k, _):
        for d in range(D // LANES):
            gi_v[k, pl.ds(d * LANES, LANES)] = z16
        return 0
    lax.fori_loop(0, KB, zb_body, 0)
    _zero_stripe(gi_v, s_sh, t)
    plsc.subcore_barrier()

    u_regs = [u_v[pl.ds(d * LANES, LANES)] for d in range(D // LANES)]

    def blk_body(b, _):
        eoff = a * EP + t * EPT + b * KB
        aoff = t * EPT + b * KB
        pltpu.sync_copy(gI_ref.at[pl.ds(eoff, KB)], gIv)
        pltpu.sync_copy(gJ_ref.at[pl.ds(eoff, KB)], gJv)
        pltpu.sync_copy(sI_ref.at[pl.ds(eoff, KB)], sIv)
        pltpu.sync_copy(attr_ref.at[pl.ds(aoff, KB)], atv)
        cg = pltpu.async_copy(gt_ref.at[gIv], gi_v, sem1)
        ch = pltpu.async_copy(ht_ref.at[gJv], hj_v, sem2)
        cg.wait()
        ch.wait()

        def kbody(k, _):
            sv = atv[k, pl.ds(0, LANES)]
            for d in range(D // LANES):
                g = gi_v[k, pl.ds(d * LANES, LANES)]
                h = hj_v[k, pl.ds(d * LANES, LANES)]
                gi_v[k, pl.ds(d * LANES, LANES)] = jnp.maximum(
                    g + h + sv * u_regs[d], 0.0)
            return 0
        lax.fori_loop(0, KB, kbody, 0)
        pltpu.sync_copy(gi_v, s_sh.at[sIv], add=True)
        return 0
    lax.fori_loop(0, NBLK, blk_body, 0)

    plsc.subcore_barrier()
    _copy_out_stripe(s_sh, out_ref, t, a * N)


@functools.partial(
    pl.kernel,
    out_type=jax.ShapeDtypeStruct((2 * N, D), jnp.float32),
    mesh=_SC_MESH,
    scratch_types=[
        pltpu.VMEM((D,), jnp.float32),
        pltpu.VMEM((KB, D), jnp.float32),
        pltpu.VMEM((KB, D), jnp.float32),
        pltpu.VMEM((KB,), jnp.int32),
        pltpu.VMEM((KB,), jnp.int32),
        pltpu.VMEM((KB,), jnp.int32),
        pltpu.VMEM((KB, LANES), jnp.float32),
        pltpu.VMEM_SHARED((SROWS, D), jnp.float32),
        pltpu.SemaphoreType.DMA,
        pltpu.SemaphoreType.DMA,
    ],
)
def _edge_sc(gt_ref, ht_ref, gI_ref, gJ_ref, sI_ref, attr_ref, u_ref,
             out_ref, u_v, gi_v, hj_v, gIv, gJv, sIv, atv, s_sh, sem1, sem2):
    _edge_kernel(gt_ref, ht_ref, gI_ref, gJ_ref, sI_ref, attr_ref, u_ref,
                 out_ref, u_v, gi_v, hj_v, gIv, gJv, sIv, atv, s_sh, sem1, sem2)


def _deg_kernel(sI_ref, out_ref, one_v, sIv, c_sh, sem1):
    a = lax.axis_index("c")
    t = lax.axis_index("s")
    z16 = jnp.zeros((LANES,), jnp.float32)
    o16 = jnp.ones((LANES,), jnp.float32)

    def zb_body(k, _):
        for d in range(D // LANES):
            one_v[k, pl.ds(d * LANES, LANES)] = z16
        return 0
    lax.fori_loop(0, KB, zb_body, 0)
    _zero_stripe(one_v, c_sh, t)

    def ob_body(k, _):
        for d in range(D // LANES):
            one_v[k, pl.ds(d * LANES, LANES)] = o16
        return 0
    lax.fori_loop(0, KB, ob_body, 0)
    plsc.subcore_barrier()

    def blk_body(b, _):
        eoff = a * EP + t * EPT + b * KB
        pltpu.sync_copy(sI_ref.at[pl.ds(eoff, KB)], sIv)
        pltpu.sync_copy(one_v, c_sh.at[sIv], add=True)
        return 0
    lax.fori_loop(0, NBLK, blk_body, 0)

    plsc.subcore_barrier()
    _copy_out_stripe(c_sh, out_ref, t, a * N)


@functools.partial(
    pl.kernel,
    out_type=jax.ShapeDtypeStruct((2 * N, D), jnp.float32),
    mesh=_SC_MESH,
    scratch_types=[
        pltpu.VMEM((KB, D), jnp.float32),
        pltpu.VMEM((KB,), jnp.int32),
        pltpu.VMEM_SHARED((SROWS, D), jnp.float32),
        pltpu.SemaphoreType.DMA,
    ],
)
def _deg_sc(sI_ref, out_ref, one_v, sIv, c_sh, sem1):
    _deg_kernel(sI_ref, out_ref, one_v, sIv, c_sh, sem1)


def kernel(nodes, edges, edge_attr, batch, enc_W, enc_b, edge_W, edge_b,
           par_W1, par_b1, par_W2, par_b2, chi_W1, chi_b1, chi_W2, chi_b2,
           fin_W1, fin_b1, fin_W2, fin_b2, conv_W, conv_b):
    f32 = jnp.float32
    src = edges[0]
    dst = edges[1]

    w_e = edge_W[:, 0]
    pWiT = par_W1[:, :D].T
    pWjT = par_W1[:, D:2 * D].T
    pu = par_W1[:, 2 * D:] @ w_e
    pc = par_W1[:, 2 * D:] @ edge_b + par_b1
    cWiT = chi_W1[:, :D].T
    cWjT = chi_W1[:, D:2 * D].T
    cu = chi_W1[:, 2 * D:] @ w_e
    cc = chi_W1[:, 2 * D:] @ edge_b + chi_b1
    WiT = jnp.stack([pWiT, cWiT])
    WjT = jnp.stack([pWjT, cWjT])
    c_stack = jnp.stack([pc, cc]).reshape(2, 1, D)
    u_flat = jnp.concatenate([pu, cu]).astype(f32)

    F1xT = fin_W1[:, :D].T
    F1iT = fin_W1[:, D:2 * D].T
    F1oT = fin_W1[:, 2 * D:].T
    F2T = fin_W2.T
    row = lambda v: v.reshape(1, -1).astype(f32)

    pad = EP - E
    i32 = jnp.int32
    padz = jnp.zeros((pad,), i32)
    padN = jnp.full((pad,), N, i32)
    gI = jnp.concatenate([dst, padz, src + N, padz])
    gJ = jnp.concatenate([src, padz, dst + N, padz])
    sI = jnp.concatenate([dst, padN, src, padN])
    attr_p = jnp.concatenate([edge_attr, jnp.zeros((pad,), f32)])
    attr16 = jnp.broadcast_to(attr_p[:, None], (EP, LANES))

    deg_out = _deg_sc(sI)
    degT = jnp.concatenate([deg_out[N:, 0:1], deg_out[:N, 0:1]], axis=1)

    x = _enc(nodes, enc_W.T, row(enc_b))

    for _ in range(NUM_IT):
        G, H = _stage_a(x, WiT, WjT, c_stack)
        S = _edge_sc(G, H, gI, gJ, sI, attr16, u_flat)
        x = _stage_b(x, S, degT, par_W2.T, row(par_b2), chi_W2.T, row(chi_b2),
                     F1xT, F1iT, F1oT, row(fin_b1), F2T, row(fin_b2))

    return _stage_c(x, conv_W.T, row(conv_b), batch.reshape(N, 1))


# speedup vs baseline: 2.9644x; 1.7198x over previous
import functools

import jax
import jax.numpy as jnp
from jax import lax
from jax.experimental import pallas as pl
from jax.experimental.pallas import tpu as pltpu
from jax.experimental.pallas import tpu_sc as plsc

N = 10000
E = 320000
D = 128
B = 64
NUM_IT = 2

NC = 2
NS = 16
LANES = 16

EPT = 20480
EP = NS * EPT
KB = 64
NBLK = EPT // KB
SROWS = 10112


def _enc_body(x_ref, w_ref, b_ref, o_ref):
    o_ref[...] = jnp.dot(x_ref[...], w_ref[...],
                         preferred_element_type=jnp.float32) + b_ref[...]


def _enc(nodes, encWT, enc_b2):
    bn = 2000
    return pl.pallas_call(
        _enc_body,
        grid=(N // bn,),
        in_specs=[
            pl.BlockSpec((bn, D), lambda i: (i, 0)),
            pl.BlockSpec((D, D), lambda i: (0, 0)),
            pl.BlockSpec((1, D), lambda i: (0, 0)),
        ],
        out_specs=pl.BlockSpec((bn, D), lambda i: (i, 0)),
        out_shape=jax.ShapeDtypeStruct((N, D), jnp.float32),
    )(nodes, encWT, enc_b2)


def _stage_a_body(x_ref, wi_ref, wj_ref, c_ref, g_ref, h_ref):
    xb = x_ref[...]
    g_ref[0] = jnp.dot(xb, wi_ref[0], preferred_element_type=jnp.float32) + c_ref[0]
    h_ref[0] = jnp.dot(xb, wj_ref[0], preferred_element_type=jnp.float32)


def _stage_a(x, WiT, WjT, c_stack):
    bn = 2000
    G, H = pl.pallas_call(
        _stage_a_body,
        grid=(2, N // bn),
        in_specs=[
            pl.BlockSpec((bn, D), lambda a, i: (i, 0)),
            pl.BlockSpec((1, D, D), lambda a, i: (a, 0, 0)),
            pl.BlockSpec((1, D, D), lambda a, i: (a, 0, 0)),
            pl.BlockSpec((1, 1, D), lambda a, i: (a, 0, 0)),
        ],
        out_specs=[
            pl.BlockSpec((1, bn, D), lambda a, i: (a, i, 0)),
            pl.BlockSpec((1, bn, D), lambda a, i: (a, i, 0)),
        ],
        out_shape=[
            jax.ShapeDtypeStruct((2, N, D), jnp.float32),
            jax.ShapeDtypeStruct((2, N, D), jnp.float32),
        ],
    )(x, WiT, WjT, c_stack)
    return G.reshape(2 * N, D), H.reshape(2 * N, D)


def _stage_b_body(x_ref, sp_ref, sc_ref, deg_ref,
                  pw2_ref, pb2_ref, cw2_ref, cb2_ref,
                  f1x_ref, f1i_ref, f1o_ref, fb1_ref, f2_ref, fb2_ref,
                  o_ref):
    ds_ = deg_ref[:, 0:1]
    dd_ = deg_ref[:, 1:2]
    dinv = jnp.where(ds_ > 0, 1.0 / ds_, 0.0)
    fi = dinv * (jnp.dot(sp_ref[...], pw2_ref[...],
                         preferred_element_type=jnp.float32) + dd_ * pb2_ref[...])
    fo = dinv * (jnp.dot(sc_ref[...], cw2_ref[...],
                         preferred_element_type=jnp.float32) + ds_ * cb2_ref[...])
    xb = x_ref[...]
    h1 = jnp.dot(xb, f1x_ref[...], preferred_element_type=jnp.float32)
    h1 = h1 + jnp.dot(fi, f1i_ref[...], preferred_element_type=jnp.float32)
    h1 = h1 + jnp.dot(fo, f1o_ref[...], preferred_element_type=jnp.float32)
    h1 = jnp.maximum(h1 + fb1_ref[...], 0.0)
    o_ref[...] = xb + jnp.dot(h1, f2_ref[...],
                              preferred_element_type=jnp.float32) + fb2_ref[...]


def _stage_b(x, S, degT, pW2T, pb2, cW2T, cb2, F1xT, F1iT, F1oT, fb1, F2T, fb2):
    bn = 1000
    nb = N // bn
    full = lambda r, c: pl.BlockSpec((r, c), lambda i: (0, 0))
    return pl.pallas_call(
        _stage_b_body,
        grid=(nb,),
        in_specs=[
            pl.BlockSpec((bn, D), lambda i: (i, 0)),
            pl.BlockSpec((bn, D), lambda i: (i, 0)),
            pl.BlockSpec((bn, D), lambda i: (nb + i, 0)),
            pl.BlockSpec((bn, 2), lambda i: (i, 0)),
            full(D, D), full(1, D), full(D, D), full(1, D),
            full(D, 2 * D), full(D, 2 * D), full(D, 2 * D), full(1, 2 * D),
            full(2 * D, D), full(1, D),
        ],
        out_specs=pl.BlockSpec((bn, D), lambda i: (i, 0)),
        out_shape=jax.ShapeDtypeStruct((N, D), jnp.float32),
    )(x, S, S, degT, pW2T, pb2, cW2T, cb2, F1xT, F1iT, F1oT, fb1, F2T, fb2)


def _stage_c_body(x_ref, w_ref, b_ref, batch_ref, o_ref, gmax_s, gsum_s, cnt_s):
    i = pl.program_id(0)
    nb = pl.num_programs(0)

    @pl.when(i == 0)
    def _init():
        gmax_s[...] = jnp.full_like(gmax_s[...], -1e30)
        gsum_s[...] = jnp.zeros_like(gsum_s[...])
        cnt_s[...] = jnp.zeros_like(cnt_s[...])

    y = jnp.dot(x_ref[...], w_ref[...],
                preferred_element_type=jnp.float32) + b_ref[...]
    bb = batch_ref[...]
    seg = lax.broadcasted_iota(jnp.int32, (1, B), 1)
    onehot = (bb == seg).astype(jnp.float32)
    gsum_s[...] += lax.dot_general(onehot, y, (((0,), (0,)), ((), ())),
                                   preferred_element_type=jnp.float32)
    ones = jnp.ones_like(bb, jnp.float32)
    cnt_s[...] += lax.dot_general(onehot, ones, (((0,), (0,)), ((), ())),
                                  preferred_element_type=jnp.float32)
    bias = (onehot - 1.0) * 1e30
    for s in range(B):
        m = jnp.max(y + bias[:, s:s + 1], axis=0, keepdims=True)
        gmax_s[s:s + 1, :] = jnp.maximum(gmax_s[s:s + 1, :], m)

    @pl.when(i == nb - 1)
    def _fin():
        o_ref[:, :2 * D] = gmax_s[...]
        o_ref[:, 2 * D:] = gsum_s[...] / jnp.maximum(cnt_s[...], 1.0)


def _stage_c(x, convWT, conv_b2, batch2):
    bn = 1000
    return pl.pallas_call(
        _stage_c_body,
        grid=(N // bn,),
        in_specs=[
            pl.BlockSpec((bn, D), lambda i: (i, 0)),
            pl.BlockSpec((D, 2 * D), lambda i: (0, 0)),
            pl.BlockSpec((1, 2 * D), lambda i: (0, 0)),
            pl.BlockSpec((bn, 1), lambda i: (i, 0)),
        ],
        out_specs=pl.BlockSpec((B, 4 * D), lambda i: (0, 0)),
        out_shape=jax.ShapeDtypeStruct((B, 4 * D), jnp.float32),
        scratch_shapes=[
            pltpu.VMEM((B, 2 * D), jnp.float32),
            pltpu.VMEM((B, 2 * D), jnp.float32),
            pltpu.VMEM((B, 1), jnp.float32),
        ],
        compiler_params=pltpu.CompilerParams(
            dimension_semantics=("arbitrary",)),
    )(x, convWT, conv_b2, batch2)


_SC_MESH = plsc.VectorSubcoreMesh(core_axis_name="c", subcore_axis_name="s",
                                  num_cores=NC, num_subcores=NS)


_RPT = SROWS // NS


def _zero_stripe_k(src_v, sh, t, kb):
    _zfull, _zrem = _RPT // kb, _RPT % kb
    for r in range(_zfull):
        pltpu.sync_copy(src_v, sh.at[pl.ds(t * _RPT + r * kb, kb)])
    if _zrem:
        pltpu.sync_copy(src_v.at[pl.ds(0, _zrem)],
                        sh.at[pl.ds(t * _RPT + _zfull * kb, _zrem)])


def _zero_stripe(src_v, sh, t):
    _zero_stripe_k(src_v, sh, t, KB)


def _copy_out_stripe(sh, out_ref, t, base):
    cw = 632
    rem = N - (NS - 1) * cw

    @pl.when(t < NS - 1)
    def _cp():
        pltpu.sync_copy(sh.at[pl.ds(t * cw, cw)],
                        out_ref.at[pl.ds(base + t * cw, cw)])

    @pl.when(t == NS - 1)
    def _cp_last():
        pltpu.sync_copy(sh.at[pl.ds((NS - 1) * cw, rem)],
                        out_ref.at[pl.ds(base + (NS - 1) * cw, rem)])


SB = 4
NSB = NBLK // SB


def _edge_kernel(gt_ref, ht_ref, idx_ref, attr_ref, u_ref, out_ref,
                 u_v, gi0, gi1, hj0, hj1, ix0, ix1, at0, at1,
                 s_sh, sg0, sg1, sh0, sh1, ss0, ss1, si0, si1):
    a = lax.axis_index("c")
    t = lax.axis_index("s")
    gi = (gi0, gi1)
    hj = (hj0, hj1)
    ix = (ix0, ix1)
    at = (at0, at1)
    sg = (sg0, sg1)
    sh = (sh0, sh1)
    ss = (ss0, ss1)
    si = (si0, si1)

    pltpu.sync_copy(u_ref.at[pl.ds(a * D, D)], u_v)
    z16 = jnp.zeros((LANES,), jnp.float32)

    def zb_body(k, _):
        for d in range(D // LANES):
            gi0[k, pl.ds(d * LANES, LANES)] = z16
        return 0
    lax.fori_loop(0, KB, zb_body, 0)
    _zero_stripe(gi0, s_sh, t)
    plsc.subcore_barrier()

    u_regs = [u_v[pl.ds(d * LANES, LANES)] for d in range(D // LANES)]
    slab0 = (a * NS + t) * NSB
    aoff0 = t * EPT

    def start_gather(rowI, rowJ, p):
        pltpu.async_copy(gt_ref.at[rowI], gi[p], sg[p])
        pltpu.async_copy(ht_ref.at[rowJ], hj[p], sh[p])

    def wait_gather(p):
        pltpu.make_async_copy(gt_ref.at[ix0.at[0, 0]], gi[p], sg[p]).wait()
        pltpu.make_async_copy(ht_ref.at[ix0.at[SB, 0]], hj[p], sh[p]).wait()

    def wait_scatter(p):
        pltpu.make_async_copy(gi[p], s_sh.at[ix0.at[2 * SB, 0]], ss[p]).wait()

    def compute(p, sp, g):
        def kbody(k8, _):
            for e in range(8):
                k = k8 * 8 + e
                sv = at[sp][g * (KB // 8) + k8, pl.ds(e * LANES, LANES)]
                for d in range(D // LANES):
                    gv = gi[p][k, pl.ds(d * LANES, LANES)]
                    hv = hj[p][k, pl.ds(d * LANES, LANES)]
                    gi[p][k, pl.ds(d * LANES, LANES)] = jnp.maximum(
                        gv + hv + sv * u_regs[d], 0.0)
            return 0
        lax.fori_loop(0, KB // 8, kbody, 0)

    pltpu.sync_copy(idx_ref.at[slab0], ix0)
    pltpu.sync_copy(attr_ref.at[pl.ds(pl.multiple_of(aoff0 // 8, 8), SB * KB // 8)], at0)
    start_gather(ix0.at[0, 0], ix0.at[SB, 0], 0)

    def pair_body(m, _):
        for sp in range(2):
            s = 2 * m + sp
            ixc, ixn = ix[sp], ix[1 - sp]
            atn = at[1 - sp]

            for g in range(SB):
                b = s * SB + g
                p = g & 1
                wait_gather(p)
                if g == 0:
                    @pl.when(b >= 1)
                    def _ws0():
                        wait_scatter(1 - p)

                    @pl.when(s + 1 < NSB)
                    def _prefetch():
                        pltpu.async_copy(idx_ref.at[slab0 + s + 1], ixn,
                                         si[1 - sp])
                        pltpu.async_copy(
                            attr_ref.at[pl.ds(pl.multiple_of(
                                (aoff0 + (s + 1) * SB * KB) // 8, 8),
                                SB * KB // 8)],
                            atn, si[1 - sp])
                    start_gather(ixc.at[1, 0], ixc.at[SB + 1, 0], 1 - p)
                elif g < SB - 1:
                    wait_scatter(1 - p)
                    start_gather(ixc.at[g + 1, 0], ixc.at[SB + g + 1, 0], 1 - p)
                else:
                    @pl.when(s + 1 < NSB)
                    def _nxt():
                        pltpu.make_async_copy(idx_ref.at[slab0 + s + 1], ixn,
                                              si[1 - sp]).wait()
                        pltpu.make_async_copy(
                            attr_ref.at[pl.ds(pl.multiple_of(
                                (aoff0 + (s + 1) * SB * KB) // 8, 8),
                                SB * KB // 8)],
                            atn, si[1 - sp]).wait()
                        wait_scatter(1 - p)
                        start_gather(ixn.at[0, 0], ixn.at[SB, 0], 1 - p)
                compute(p, sp, g)
                pltpu.async_copy(gi[p], s_sh.at[ixc.at[2 * SB + g, 0]], ss[p],
                                 add=True)
        return 0
    lax.fori_loop(0, NSB // 2, pair_body, 0)
    wait_scatter(0)
    wait_scatter(1)

    plsc.subcore_barrier()
    _copy_out_stripe(s_sh, out_ref, t, a * N)


@functools.partial(
    pl.kernel,
    out_type=jax.ShapeDtypeStruct((2 * N, D), jnp.float32),
    mesh=_SC_MESH,
    scratch_types=[
        pltpu.VMEM((D,), jnp.float32),
        pltpu.VMEM((KB, D), jnp.float32),
        pltpu.VMEM((KB, D), jnp.float32),
        pltpu.VMEM((KB, D), jnp.float32),
        pltpu.VMEM((KB, D), jnp.float32),
        pltpu.VMEM((3 * SB, 1, KB), jnp.int32),
        pltpu.VMEM((3 * SB, 1, KB), jnp.int32),
        pltpu.VMEM((SB * KB // 8, D), jnp.float32),
        pltpu.VMEM((SB * KB // 8, D), jnp.float32),
        pltpu.VMEM_SHARED((SROWS, D), jnp.float32),
        pltpu.SemaphoreType.DMA, pltpu.SemaphoreType.DMA,
        pltpu.SemaphoreType.DMA, pltpu.SemaphoreType.DMA,
        pltpu.SemaphoreType.DMA, pltpu.SemaphoreType.DMA,
        pltpu.SemaphoreType.DMA, pltpu.SemaphoreType.DMA,
    ],
)
def _edge_sc(gt_ref, ht_ref, idx_ref, attr_ref, u_ref, out_ref,
             u_v, gi0, gi1, hj0, hj1, ix0, ix1, at0, at1,
             s_sh, sg0, sg1, sh0, sh1, ss0, ss1, si0, si1):
    _edge_kernel(gt_ref, ht_ref, idx_ref, attr_ref, u_ref, out_ref,
                 u_v, gi0, gi1, hj0, hj1, ix0, ix1, at0, at1,
                 s_sh, sg0, sg1, sh0, sh1, ss0, ss1, si0, si1)


KBD = 128
NBLKD = EPT // KBD


def _deg_kernel(sI_ref, out_ref, one_v, ixd, c_sh, ssd, sid0, sid1, sid2, sid3):
    a = lax.axis_index("c")
    t = lax.axis_index("s")
    sid = (sid0, sid1, sid2, sid3)
    z16 = jnp.zeros((LANES,), jnp.float32)
    o16 = jnp.ones((LANES,), jnp.float32)

    def zb_body(k, _):
        for d in range(D // LANES):
            one_v[k, pl.ds(d * LANES, LANES)] = z16
        return 0
    lax.fori_loop(0, KBD, zb_body, 0)
    _zero_stripe_k(one_v, c_sh, t, KBD)

    def ob_body(k, _):
        for d in range(D // LANES):
            one_v[k, pl.ds(d * LANES, LANES)] = o16
        return 0
    lax.fori_loop(0, KBD, ob_body, 0)
    plsc.subcore_barrier()

    base = a * EP + t * EPT
    pltpu.sync_copy(sI_ref.at[pl.ds(base, KBD)], ixd.at[0, 0])

    def wait_sc():
        pltpu.make_async_copy(one_v, c_sh.at[ixd.at[0, 0]], ssd).wait()

    def quad_body(m, _):
        for q in range(4):
            b = 4 * m + q
            qn = (q + 1) % 4

            @pl.when(b >= 2)
            def _ws():
                wait_sc()

            @pl.when(b + 1 < NBLKD)
            def _pf():
                pltpu.async_copy(sI_ref.at[pl.ds(base + (b + 1) * KBD, KBD)],
                                 ixd.at[qn, 0], sid[qn])
            pltpu.async_copy(one_v, c_sh.at[ixd.at[q, 0]], ssd, add=True)

            @pl.when(b + 1 < NBLKD)
            def _wf():
                pltpu.make_async_copy(
                    sI_ref.at[pl.ds(base + (b + 1) * KBD, KBD)],
                    ixd.at[qn, 0], sid[qn]).wait()
        return 0
    lax.fori_loop(0, NBLKD // 4, quad_body, 0)
    wait_sc()
    wait_sc()

    plsc.subcore_barrier()
    _copy_out_stripe(c_sh, out_ref, t, a * N)


@functools.partial(
    pl.kernel,
    out_type=jax.ShapeDtypeStruct((2 * N, D), jnp.float32),
    mesh=_SC_MESH,
    scratch_types=[
        pltpu.VMEM((KBD, D), jnp.float32),
        pltpu.VMEM((4, 1, KBD), jnp.int32),
        pltpu.VMEM_SHARED((SROWS, D), jnp.float32),
        pltpu.SemaphoreType.DMA,
        pltpu.SemaphoreType.DMA, pltpu.SemaphoreType.DMA,
        pltpu.SemaphoreType.DMA, pltpu.SemaphoreType.DMA,
    ],
)
def _deg_sc(sI_ref, out_ref, one_v, ixd, c_sh, ssd, sid0, sid1, sid2, sid3):
    _deg_kernel(sI_ref, out_ref, one_v, ixd, c_sh, ssd, sid0, sid1, sid2, sid3)


def kernel(nodes, edges, edge_attr, batch, enc_W, enc_b, edge_W, edge_b,
           par_W1, par_b1, par_W2, par_b2, chi_W1, chi_b1, chi_W2, chi_b2,
           fin_W1, fin_b1, fin_W2, fin_b2, conv_W, conv_b):
    f32 = jnp.float32
    src = edges[0]
    dst = edges[1]

    w_e = edge_W[:, 0]
    pWiT = par_W1[:, :D].T
    pWjT = par_W1[:, D:2 * D].T
    pu = par_W1[:, 2 * D:] @ w_e
    pc = par_W1[:, 2 * D:] @ edge_b + par_b1
    cWiT = chi_W1[:, :D].T
    cWjT = chi_W1[:, D:2 * D].T
    cu = chi_W1[:, 2 * D:] @ w_e
    cc = chi_W1[:, 2 * D:] @ edge_b + chi_b1
    WiT = jnp.stack([pWiT, cWiT])
    WjT = jnp.stack([pWjT, cWjT])
    c_stack = jnp.stack([pc, cc]).reshape(2, 1, D)
    u_flat = jnp.concatenate([pu, cu]).astype(f32)

    F1xT = fin_W1[:, :D].T
    F1iT = fin_W1[:, D:2 * D].T
    F1oT = fin_W1[:, 2 * D:].T
    F2T = fin_W2.T
    row = lambda v: v.reshape(1, -1).astype(f32)

    pad = EP - E
    i32 = jnp.int32
    padz = jnp.zeros((pad,), i32)
    padN = jnp.full((pad,), N, i32)
    gI = jnp.concatenate([dst, padz, src + N, padz])
    gJ = jnp.concatenate([src, padz, dst + N, padz])
    sI = jnp.concatenate([dst, padN, src, padN])
    IDX = jnp.stack([gI.reshape(2, NS, NSB, SB, KB),
                     gJ.reshape(2, NS, NSB, SB, KB),
                     sI.reshape(2, NS, NSB, SB, KB)], axis=3)
    IDX = IDX.reshape(2 * NS * NSB, 3 * SB, 1, KB)
    attr_p = jnp.concatenate([edge_attr, jnp.zeros((pad,), f32)])
    attr16 = jnp.broadcast_to(attr_p[:, None], (EP, LANES)).reshape(EP // 8, D)

    deg_out = _deg_sc(sI)
    degT = jnp.concatenate([deg_out[N:, 0:1], deg_out[:N, 0:1]], axis=1)

    x = _enc(nodes, enc_W.T, row(enc_b))

    for _ in range(NUM_IT):
        G, H = _stage_a(x, WiT, WjT, c_stack)
        S = _edge_sc(G, H, IDX, attr16, u_flat)
        x = _stage_b(x, S, degT, par_W2.T, row(par_b2), chi_W2.T, row(chi_b2),
                     F1xT, F1iT, F1oT, row(fin_b1), F2T, row(fin_b2))

    return _stage_c(x, conv_W.T, row(conv_b), batch.reshape(N, 1))
